# baseline (speedup 1.0000x reference)
import functools

import jax
import jax.numpy as jnp
from jax.experimental import pallas as pl
from jax.experimental.pallas import tpu as pltpu

_EPS = 1e-5


def _sample_col_kernel(flf3_ref, flf2_ref, col_ref, *, t):
    m = jnp.max(flf3_ref[...], axis=0)
    s = jnp.sum(m)
    mn = m / s
    R, K = m.shape
    ku = jax.lax.broadcasted_iota(jnp.int32, (K, K), 0)
    kv = jax.lax.broadcasted_iota(jnp.int32, (K, K), 1)
    upper = (ku <= kv).astype(jnp.float32)
    rowcum = jnp.dot(mn, upper, preferred_element_type=jnp.float32)
    ru = jax.lax.broadcasted_iota(jnp.int32, (R, R), 0)
    rv = jax.lax.broadcasted_iota(jnp.int32, (R, R), 1)
    strict_lower = (rv < ru).astype(jnp.float32)
    rowtot = jnp.sum(mn, axis=1, keepdims=True)
    prev = jnp.dot(strict_lower, rowtot, preferred_element_type=jnp.float32)
    cdf = rowcum + prev
    cdf_i = (cdf * jnp.float32(t)).astype(jnp.int32)
    sentinel = jnp.int32(jnp.iinfo(jnp.int32).max)
    cur = jnp.min(jnp.where(cdf_i >= 0, cdf_i, sentinel))
    lin = (jax.lax.broadcasted_iota(jnp.int32, (R, K), 0) * K
           + jax.lax.broadcasted_iota(jnp.int32, (R, K), 1))
    big = jnp.int32(1 << 30)
    hit = jnp.min(jnp.where(cdf_i == cur, lin, big))
    first_idx = jnp.where(hit == big, jnp.int32(0), hit)
    lin2 = jax.lax.broadcasted_iota(jnp.int32, (R * K, 1), 0)
    onehot = (lin2 == first_idx).astype(jnp.float32)
    col_ref[...] = jnp.dot(flf2_ref[...], onehot,
                           preferred_element_type=jnp.float32)


def _gn_relu(a, gamma, beta, groups):
    c, tt = a.shape
    gs = c // groups
    rs = jnp.sum(a, axis=1, keepdims=True)
    rq = jnp.sum(a * a, axis=1, keepdims=True)
    gi = jax.lax.broadcasted_iota(jnp.int32, (groups, c), 0)
    gc = jax.lax.broadcasted_iota(jnp.int32, (groups, c), 1) // gs
    gind = (gi == gc).astype(jnp.float32)
    ci = jax.lax.broadcasted_iota(jnp.int32, (c, groups), 0) // gs
    cg = jax.lax.broadcasted_iota(jnp.int32, (c, groups), 1)
    gind_t = (ci == cg).astype(jnp.float32)
    cnt = jnp.float32(gs * tt)
    gmean = jnp.dot(gind, rs, preferred_element_type=jnp.float32) / cnt
    gsq = jnp.dot(gind, rq, preferred_element_type=jnp.float32) / cnt
    gvar = gsq - gmean * gmean
    mean_c = jnp.dot(gind_t, gmean, preferred_element_type=jnp.float32)
    var_c = jnp.dot(gind_t, gvar, preferred_element_type=jnp.float32)
    xn = (a - mean_c) * jax.lax.rsqrt(var_c + _EPS)
    return jnp.maximum(xn * gamma + beta, 0.0)


def _main_kernel(x_ref, col_ref, wcur_ref, bcur_ref, gcur_ref, becur_ref,
                 wlr_ref, blr_ref, glr_ref, belr_ref,
                 wps_ref, wpf_ref, wpm_ref, bprop_ref, gprop_ref, beprop_ref,
                 mixed_ref, feat_ref):
    x = x_ref[...]
    a1 = jnp.dot(wcur_ref[...], x,
                 preferred_element_type=jnp.float32) + bcur_ref[...]
    fm_short = _gn_relu(a1, gcur_ref[...], becur_ref[...], 32)
    a2 = jnp.dot(wlr_ref[...], x,
                 preferred_element_type=jnp.float32) + blr_ref[...]
    feat = _gn_relu(a2, glr_ref[...], belr_ref[...], 32)
    feat_ref[...] = feat
    v = jnp.dot(wps_ref[...], col_ref[...],
                preferred_element_type=jnp.float32) + bprop_ref[...]
    y = (jnp.dot(wpf_ref[...], feat.astype(jnp.bfloat16),
                 preferred_element_type=jnp.float32)
         + jnp.dot(wpm_ref[...], fm_short.astype(jnp.bfloat16),
                   preferred_element_type=jnp.float32)
         + v)
    mixed_ref[...] = _gn_relu(y, gprop_ref[...], beprop_ref[...], 32)


def kernel(feature, frame_level_feature, w_cur, b_cur, g_cur, be_cur,
           w_lr, b_lr, g_lr, be_lr, w_prop, b_prop, g_prop, be_prop):
    x = feature[0]
    flf2 = frame_level_feature[0]
    c, t = x.shape
    T = flf2.shape[1]
    R = 32
    K = T // R
    flf3 = flf2.reshape(c, R, K)

    col = pl.pallas_call(
        functools.partial(_sample_col_kernel, t=t),
        out_shape=jax.ShapeDtypeStruct((c, 1), jnp.float32),
        compiler_params=pltpu.CompilerParams(vmem_limit_bytes=100 * 2**20),
    )(flf3, flf2)

    pc = w_cur.shape[0]
    pc2 = w_lr.shape[0]
    wps = w_prop[:, :pc]
    wpf = w_prop[:, pc:pc + pc2]
    wpm = w_prop[:, pc + pc2:]

    mixed, feat = pl.pallas_call(
        _main_kernel,
        out_shape=[
            jax.ShapeDtypeStruct((w_prop.shape[0], t), jnp.float32),
            jax.ShapeDtypeStruct((pc2, t), jnp.float32),
        ],
        compiler_params=pltpu.CompilerParams(vmem_limit_bytes=100 * 2**20),
    )(x.astype(jnp.bfloat16), col, w_cur.astype(jnp.bfloat16),
      b_cur.reshape(-1, 1), g_cur.reshape(-1, 1),
      be_cur.reshape(-1, 1), w_lr.astype(jnp.bfloat16), b_lr.reshape(-1, 1),
      g_lr.reshape(-1, 1), be_lr.reshape(-1, 1), wps,
      wpf.astype(jnp.bfloat16), wpm.astype(jnp.bfloat16),
      b_prop.reshape(-1, 1), g_prop.reshape(-1, 1), be_prop.reshape(-1, 1))

    return (mixed[None], feat[None])


# speedup vs baseline: 1.3962x; 1.3962x over previous
import functools

import jax
import jax.numpy as jnp
from jax.experimental import pallas as pl
from jax.experimental.pallas import tpu as pltpu

_EPS = 1e-5


def _sample_col_kernel(flf_ref, col_ref, *, t, R, K):
    flf = flf_ref[...]
    m1 = jnp.max(flf, axis=0, keepdims=True)
    m = jnp.concatenate([m1[:, i * K:(i + 1) * K] for i in range(R)], axis=0)
    s = jnp.sum(m)
    mn = m / s
    ku = jax.lax.broadcasted_iota(jnp.int32, (K, K), 0)
    kv = jax.lax.broadcasted_iota(jnp.int32, (K, K), 1)
    upper = (ku <= kv).astype(jnp.float32)
    rowcum = jnp.dot(mn, upper, preferred_element_type=jnp.float32)
    ru = jax.lax.broadcasted_iota(jnp.int32, (R, R), 0)
    rv = jax.lax.broadcasted_iota(jnp.int32, (R, R), 1)
    strict_lower = (rv < ru).astype(jnp.float32)
    rowtot = jnp.sum(mn, axis=1, keepdims=True)
    prev = jnp.dot(strict_lower, rowtot, preferred_element_type=jnp.float32)
    cdf = rowcum + prev
    cdf_i = (cdf * jnp.float32(t)).astype(jnp.int32)
    sentinel = jnp.int32(jnp.iinfo(jnp.int32).max)
    cur = jnp.min(jnp.where(cdf_i >= 0, cdf_i, sentinel))
    lin = (jax.lax.broadcasted_iota(jnp.int32, (R, K), 0) * K
           + jax.lax.broadcasted_iota(jnp.int32, (R, K), 1))
    big = jnp.int32(1 << 30)
    hit = jnp.min(jnp.where(cdf_i == cur, lin, big))
    first_idx = jnp.where(hit == big, jnp.int32(0), hit)
    lin2 = jax.lax.broadcasted_iota(jnp.int32, (R * K, 1), 0)
    onehot = (lin2 == first_idx).astype(jnp.float32)
    col_ref[...] = jnp.dot(flf, onehot, preferred_element_type=jnp.float32)


def _gn_relu(a, gamma, beta, groups):
    c, tt = a.shape
    gs = c // groups
    rs = jnp.sum(a, axis=1, keepdims=True)
    rq = jnp.sum(a * a, axis=1, keepdims=True)
    gi = jax.lax.broadcasted_iota(jnp.int32, (groups, c), 0)
    gc = jax.lax.broadcasted_iota(jnp.int32, (groups, c), 1) // gs
    gind = (gi == gc).astype(jnp.float32)
    ci = jax.lax.broadcasted_iota(jnp.int32, (c, groups), 0) // gs
    cg = jax.lax.broadcasted_iota(jnp.int32, (c, groups), 1)
    gind_t = (ci == cg).astype(jnp.float32)
    cnt = jnp.float32(gs * tt)
    gmean = jnp.dot(gind, rs, preferred_element_type=jnp.float32) / cnt
    gsq = jnp.dot(gind, rq, preferred_element_type=jnp.float32) / cnt
    gvar = gsq - gmean * gmean
    mean_c = jnp.dot(gind_t, gmean, preferred_element_type=jnp.float32)
    var_c = jnp.dot(gind_t, gvar, preferred_element_type=jnp.float32)
    xn = (a - mean_c) * jax.lax.rsqrt(var_c + _EPS)
    return jnp.maximum(xn * gamma + beta, 0.0)


def _main_kernel(x_ref, col_ref, wcur_ref, bcur_ref, gcur_ref, becur_ref,
                 wlr_ref, blr_ref, glr_ref, belr_ref,
                 wps_ref, wpf_ref, wpm_ref, bprop_ref, gprop_ref, beprop_ref,
                 mixed_ref, feat_ref):
    x = x_ref[...]
    a1 = jnp.dot(wcur_ref[...], x,
                 preferred_element_type=jnp.float32) + bcur_ref[...]
    fm_short = _gn_relu(a1, gcur_ref[...], becur_ref[...], 32)
    a2 = jnp.dot(wlr_ref[...], x,
                 preferred_element_type=jnp.float32) + blr_ref[...]
    feat = _gn_relu(a2, glr_ref[...], belr_ref[...], 32)
    feat_ref[...] = feat
    v = jnp.dot(wps_ref[...], col_ref[...],
                preferred_element_type=jnp.float32) + bprop_ref[...]
    y = (jnp.dot(wpf_ref[...], feat, preferred_element_type=jnp.float32)
         + jnp.dot(wpm_ref[...], fm_short, preferred_element_type=jnp.float32)
         + v)
    mixed_ref[...] = _gn_relu(y, gprop_ref[...], beprop_ref[...], 32)


def kernel(feature, frame_level_feature, w_cur, b_cur, g_cur, be_cur,
           w_lr, b_lr, g_lr, be_lr, w_prop, b_prop, g_prop, be_prop):
    x = feature[0]
    flf2 = frame_level_feature[0]
    c, t = x.shape
    T = flf2.shape[1]
    R = 32
    K = T // R

    col = pl.pallas_call(
        functools.partial(_sample_col_kernel, t=t, R=R, K=K),
        out_shape=jax.ShapeDtypeStruct((c, 1), jnp.float32),
        compiler_params=pltpu.CompilerParams(vmem_limit_bytes=100 * 2**20),
    )(flf2)

    pc = w_cur.shape[0]
    pc2 = w_lr.shape[0]
    wps = w_prop[:, :pc]
    wpf = w_prop[:, pc:pc + pc2]
    wpm = w_prop[:, pc + pc2:]

    mixed, feat = pl.pallas_call(
        _main_kernel,
        out_shape=[
            jax.ShapeDtypeStruct((w_prop.shape[0], t), jnp.float32),
            jax.ShapeDtypeStruct((pc2, t), jnp.float32),
        ],
        compiler_params=pltpu.CompilerParams(vmem_limit_bytes=100 * 2**20),
    )(x, col, w_cur, b_cur.reshape(-1, 1), g_cur.reshape(-1, 1),
      be_cur.reshape(-1, 1), w_lr, b_lr.reshape(-1, 1), g_lr.reshape(-1, 1),
      be_lr.reshape(-1, 1), wps, wpf, wpm, b_prop.reshape(-1, 1),
      g_prop.reshape(-1, 1), be_prop.reshape(-1, 1))

    return (mixed[None], feat[None])
